# Initial kernel scaffold; baseline (speedup 1.0000x reference)
import jax
import jax.numpy as jnp
from jax.experimental import pallas as pl


def kernel(blocks, targets, table):
    raise NotImplementedError("write your pallas kernel here")


# speedup vs baseline: 7.5794x; 7.5794x over previous
import functools

import jax
import jax.numpy as jnp
from jax import lax
from jax.experimental import pallas as pl
from jax.experimental.pallas import tpu as pltpu
from jax.experimental.pallas import tpu_sc as plsc

V = 8192
N_TOK = 256 * 128


_ROWS_PER_BLK = 256
_N_BLKS = V // _ROWS_PER_BLK


def _lse_body(x_ref, o_ref):
    x = x_ref[...]
    m = jnp.max(x, axis=1)
    s = jnp.sum(jnp.exp(x - m[:, None]), axis=1)
    o_ref[...] = (m + jnp.log(s)).reshape(1, 1, _ROWS_PER_BLK)


def _row_logsumexp(table):
    out = pl.pallas_call(
        _lse_body,
        grid=(_N_BLKS,),
        in_specs=[pl.BlockSpec((_ROWS_PER_BLK, V), lambda i: (i, 0))],
        out_specs=pl.BlockSpec((1, 1, _ROWS_PER_BLK), lambda i: (i, 0, 0)),
        out_shape=jax.ShapeDtypeStruct((_N_BLKS, 1, _ROWS_PER_BLK), jnp.float32),
    )(table)
    return out.reshape(V)


_NC, _NS, _L = 2, 16, 16
_NW = _NC * _NS
_BPW = N_TOK // _NW
_CH = 128
_NCH = _BPW // _CH

_sc_mesh = plsc.VectorSubcoreMesh(core_axis_name="c", subcore_axis_name="s")


@functools.partial(
    pl.kernel,
    out_type=jax.ShapeDtypeStruct((_NW * _L,), jnp.float32),
    mesh=_sc_mesh,
    scratch_types=[
        pltpu.VMEM((_NCH, _CH), jnp.int32),
        pltpu.VMEM((_NCH, _CH), jnp.int32),
        pltpu.VMEM((_BPW,), jnp.float32),
        pltpu.VMEM((_BPW,), jnp.float32),
        pltpu.VMEM((_L,), jnp.float32),
        pltpu.SemaphoreType.DMA,
    ],
)
def _sc_gather(flat_idx_hbm, blocks_hbm, table_flat_hbm, logz_hbm, out_hbm,
               idx_v, blk_v, vals_v, lz_v, part_v, sem):
    wid = lax.axis_index("s") * _NC + lax.axis_index("c")

    pltpu.sync_copy(flat_idx_hbm.at[wid], idx_v)
    pltpu.sync_copy(blocks_hbm.at[wid], blk_v)
    copies = []
    for j in range(_NCH):
        copies.append(
            pltpu.async_copy(table_flat_hbm.at[idx_v.at[j]],
                             vals_v.at[pl.ds(j * _CH, _CH)], sem))
        copies.append(
            pltpu.async_copy(logz_hbm.at[blk_v.at[j]],
                             lz_v.at[pl.ds(j * _CH, _CH)], sem))
    for cp in copies:
        cp.wait()

    def body(i, acc):
        lz = lz_v[pl.ds(i * _L, _L)]
        tv = vals_v[pl.ds(i * _L, _L)]
        return acc + (lz - tv)

    acc = lax.fori_loop(0, _BPW // _L, body, jnp.zeros((_L,), jnp.float32))
    part_v[...] = acc
    pltpu.sync_copy(part_v, out_hbm.at[pl.ds(wid * _L, _L)])


def kernel(blocks, targets, table):
    b = blocks.reshape(-1).astype(jnp.int32)
    t = targets.reshape(-1).astype(jnp.int32)
    flat_idx = (b * V + t).reshape(_NW, _NCH, _CH)
    b_sh = b.reshape(_NW, _NCH, _CH)
    logz = _row_logsumexp(table)
    parts = _sc_gather(flat_idx, b_sh, table.reshape(V * V), logz)
    return jnp.sum(parts) / N_TOK
